# baseline (speedup 1.0000x reference)
import functools

import jax
import jax.numpy as jnp
from jax import lax
from jax.experimental import pallas as pl
from jax.experimental.pallas import tpu as pltpu
from jax.experimental.pallas import tpu_sc as plsc

N = 50000
D = 128
NC = 2
NS = 16
BE = 128
SB = 2048
CH = 8448
NCHUNK = 6
NPAD = CH * NCHUNK
PASSES = NCHUNK // NC
CPT = CH // NS
ZR = 66
BR = 1000


def _pad_edges(ei, w):
    e = ei.shape[1]
    ept = ((e + NS * 2 * SB - 1) // (NS * 2 * SB)) * 2 * SB
    pad = ept * NS - e
    src = jnp.concatenate([ei[0], jnp.zeros((pad,), jnp.int32)])
    dst = jnp.concatenate([ei[1], jnp.zeros((pad,), jnp.int32)])
    wp = jnp.concatenate([w, jnp.zeros((pad,), jnp.float32)])
    return src, dst, wp, ept


@functools.cache
def _make_segsum(ept):
    nsb = ept // SB
    npair = nsb // 2
    nblk = SB // BE
    mesh = plsc.VectorSubcoreMesh(core_axis_name="c", subcore_axis_name="s")

    @functools.partial(
        pl.kernel,
        mesh=mesh,
        compiler_params=pltpu.CompilerParams(needs_layout_passes=False),
        out_type=jax.ShapeDtypeStruct((NPAD, D), jnp.float32),
        scratch_types=[
            pltpu.VMEM_SHARED((CH, D), jnp.float32),
            pltpu.VMEM((ZR, D), jnp.float32),
            pltpu.VMEM((SB,), jnp.int32),
            pltpu.VMEM((SB,), jnp.int32),
            pltpu.VMEM((SB,), jnp.float32),
            pltpu.VMEM((SB,), jnp.int32),
            pltpu.VMEM((SB,), jnp.int32),
            pltpu.VMEM((SB,), jnp.float32),
            pltpu.VMEM((BE,), jnp.int32),
            pltpu.VMEM((BE,), jnp.int32),
            pltpu.VMEM((BE,), jnp.float32),
            pltpu.VMEM((BE,), jnp.int32),
            pltpu.VMEM((BE,), jnp.int32),
            pltpu.VMEM((BE,), jnp.float32),
            pltpu.VMEM((BE,), jnp.int32),
            pltpu.VMEM((BE,), jnp.int32),
            pltpu.VMEM((BE,), jnp.float32),
            pltpu.VMEM((BE,), jnp.int32),
            pltpu.VMEM((BE,), jnp.int32),
            pltpu.VMEM((BE,), jnp.float32),
            pltpu.VMEM((BE, D), jnp.float32),
            pltpu.VMEM((BE, D), jnp.float32),
            pltpu.SemaphoreType.DMA,
            pltpu.SemaphoreType.DMA,
            pltpu.SemaphoreType.DMA,
            pltpu.SemaphoreType.DMA,
            pltpu.SemaphoreType.DMA,
            pltpu.SemaphoreType.DMA,
        ],
    )
    def seg(h_hbm, src_hbm, dst_hbm, w_hbm, out_hbm,
            acc, zbuf, ebs0, ebd0, ebw0, ebs1, ebd1, ebw1,
            sts, std, stw, ovs, ovd, ovw,
            gs0, gd0, gw0, gs1, gd1, gw1, rows0, rows1,
            esem0, esem1, gsem0, gsem1, ssem0, ssem1):
        c = lax.axis_index("c")
        s = lax.axis_index("s")
        zero16f = jnp.zeros((16,), jnp.float32)
        zero16i = jnp.zeros((16,), jnp.int32)
        iota16 = lax.iota(jnp.int32, 16)

        def zb(i, carry):
            for u in range(8):
                zbuf[i, pl.ds(u * 16, 16)] = zero16f
            return carry

        lax.fori_loop(0, ZR, zb, 0)
        for g in range(8):
            sl = pl.ds(g * 16, 16)
            sts[sl] = zero16i
            std[sl] = zero16i
            stw[sl] = zero16f
            ovs[sl] = zero16i
            ovd[sl] = zero16i
            ovw[sl] = zero16f

        e0t = s * ept
        ebufs = ((ebs0, ebd0, ebw0, esem0), (ebs1, ebd1, ebw1, esem1))
        gsets = ((gs0, gd0, gw0, rows0, gsem0, ssem0),
                 (gs1, gd1, gw1, rows1, gsem1, ssem1))

        def load_sb(t, half):
            bs, bd, bw, sem = ebufs[half]
            e0 = e0t + t * SB
            pltpu.async_copy(src_hbm.at[pl.ds(e0, SB)], bs, sem)
            pltpu.async_copy(dst_hbm.at[pl.ds(e0, SB)], bd, sem)
            pltpu.async_copy(w_hbm.at[pl.ds(e0, SB)], bw, sem)

        def wait_sb(t, half):
            bs, bd, bw, sem = ebufs[half]
            e0 = e0t + t * SB
            pltpu.make_async_copy(src_hbm.at[pl.ds(e0, SB)], bs, sem).wait()
            pltpu.make_async_copy(dst_hbm.at[pl.ds(e0, SB)], bd, sem).wait()
            pltpu.make_async_copy(w_hbm.at[pl.ds(e0, SB)], bw, sem).wait()

        def start_gather(k):
            gs, gd, gw, rows, sem, ssem = gsets[k]
            pltpu.async_copy(h_hbm.at[gs], rows, sem)

        def wait_scatter(k):
            gs, gd, gw, rows, sem, ssem = gsets[k]
            pltpu.make_async_copy(rows, acc.at[gd], ssem).wait()

        def process(k):
            gs, gd, gw, rows, sem, ssem = gsets[k]
            pltpu.make_async_copy(h_hbm.at[gs], rows, sem).wait()

            def scale(j, carry2):
                wspl = plsc.load_gather(gw, [jnp.full((16,), j, jnp.int32)])
                for u in range(8):
                    sl2 = pl.ds(u * 16, 16)
                    rows[j, sl2] = rows[j, sl2] * wspl
                return carry2

            lax.fori_loop(0, BE, scale, 0)
            pltpu.async_copy(rows, acc.at[gd], ssem, add=True)

        def stage_to_set(k):
            gs, gd, gw, rows, sem, ssem = gsets[k]
            for g in range(8):
                sl = pl.ds(g * 16, 16)
                gs[sl] = sts[sl]
                gd[sl] = std[sl]
                gw[sl] = stw[sl]

        def ovf_to_stage():
            for g in range(8):
                sl = pl.ds(g * 16, 16)
                sts[sl] = ovs[sl]
                std[sl] = ovd[sl]
                stw[sl] = ovw[sl]

        def make_fire(k):
            o = 1 - k

            def fire_k(st):
                cnt, fp, pd0, pd1, ps0, ps1 = st
                pend_other = pd1 if k == 0 else pd0
                ps_k = ps0 if k == 0 else ps1

                @pl.when(pend_other == 1)
                def _():
                    process(o)

                @pl.when(ps_k == 1)
                def _():
                    wait_scatter(k)

                stage_to_set(k)
                start_gather(k)
                ovf_to_stage()
                one = jnp.int32(1)
                if k == 0:
                    return (cnt - BE, one, one, jnp.int32(0),
                            jnp.int32(0), ps1 | pd1)
                return (cnt - BE, jnp.int32(0), jnp.int32(0), one,
                        ps0 | pd0, jnp.int32(0))
            return fire_k

        fire0 = make_fire(0)
        fire1 = make_fire(1)

        for p in range(PASSES):
            q = c * PASSES + p
            base = q * CH
            with jax.named_scope("segzero"):
                for k in range(CPT // ZR):
                    pltpu.sync_copy(zbuf, acc.at[pl.ds(s * CPT + k * ZR, ZR)])
                plsc.subcore_barrier()

            def scan_buf(half, st):
                bs, bd, bw, _ = ebufs[half]

                def eblk(b, st2):
                    cnt = st2[0]
                    boff = b * BE
                    for g in range(8):
                        sl = pl.ds(boff + g * 16, 16)
                        dv = bd[sl]
                        sv = bs[sl]
                        wv = bw[sl]
                        inm = (dv >= base) & (dv < base + CH)
                        ones = jnp.where(inm, 1, 0).astype(jnp.int32)
                        pos = cnt + plsc.cumsum(ones) - 1
                        posm = pos & (BE - 1)
                        in_a = inm & (pos < BE)
                        in_b = inm & (pos >= BE)
                        dloc = dv - base
                        plsc.store_scatter(sts, [posm], sv, mask=in_a)
                        plsc.store_scatter(std, [posm], dloc, mask=in_a)
                        plsc.store_scatter(stw, [posm], wv, mask=in_a)
                        plsc.store_scatter(ovs, [posm], sv, mask=in_b)
                        plsc.store_scatter(ovd, [posm], dloc, mask=in_b)
                        plsc.store_scatter(ovw, [posm], wv, mask=in_b)
                        cnt = cnt + plsc.all_reduce_population_count(inm)

                    st2 = (cnt,) + st2[1:]

                    def fire(stf):
                        return lax.cond(stf[1] == 0, fire0, fire1, stf)

                    return lax.cond(jnp.max(cnt) >= BE, fire,
                                    lambda a: a, st2)

                return lax.fori_loop(0, nblk, eblk, st)

            load_sb(0, 0)

            def sbpair(t2, st):
                t0 = 2 * t2
                load_sb(t0 + 1, 1)
                wait_sb(t0, 0)
                st = scan_buf(0, st)

                @pl.when(t2 + 1 < npair)
                def _():
                    load_sb(t0 + 2, 0)

                wait_sb(t0 + 1, 1)
                return scan_buf(1, st)

            z = jnp.int32(0)
            with jax.named_scope("segscan"):
                cnt, fp, pd0, pd1, ps0, ps1 = lax.fori_loop(
                    0, npair, sbpair, (zero16i, z, z, z, z, z))

            with jax.named_scope("segflush"):
                @pl.when(pd0 == 1)
                def _():
                    process(0)

                @pl.when(pd1 == 1)
                def _():
                    process(1)

                @pl.when((ps0 | pd0) == 1)
                def _():
                    wait_scatter(0)

                @pl.when((ps1 | pd1) == 1)
                def _():
                    wait_scatter(1)

                for g in range(8):
                    sl = pl.ds(g * 16, 16)
                    lane = iota16 + g * 16
                    stw[sl] = jnp.where(lane < cnt, stw[sl], 0.0)
                stage_to_set(0)
                start_gather(0)
                process(0)
                wait_scatter(0)
            with jax.named_scope("segout"):
                plsc.subcore_barrier()
                pltpu.sync_copy(
                    acc.at[pl.ds(s * CPT, CPT)],
                    out_hbm.at[pl.ds(base + s * CPT, CPT)])
                plsc.subcore_barrier()

    return seg


def _mm(xs, ws, b, act):
    nin = len(xs)

    def body(*refs):
        in_refs = refs[:nin]
        w_refs = refs[nin:2 * nin]
        b_ref = refs[2 * nin]
        o_ref = refs[2 * nin + 1]
        acc = jnp.zeros((BR, D), jnp.float32)
        for xr, wr in zip(in_refs, w_refs):
            acc = acc + jnp.dot(xr[...], wr[...],
                                preferred_element_type=jnp.float32)
        acc = acc + b_ref[...]
        if act == "relu":
            acc = jnp.maximum(acc, 0.0)
        elif act == "lrelu":
            acc = jnp.where(acc > 0, acc, acc * 0.01)
        o_ref[...] = acc

    in_specs = (
        [pl.BlockSpec((BR, D), lambda i: (i, 0)) for _ in xs]
        + [pl.BlockSpec((D, D), lambda i: (0, 0)) for _ in ws]
        + [pl.BlockSpec((1, D), lambda i: (0, 0))]
    )
    f = pl.pallas_call(
        body,
        grid=(N // BR,),
        in_specs=in_specs,
        out_specs=pl.BlockSpec((BR, D), lambda i: (i, 0)),
        out_shape=jax.ShapeDtypeStruct((N, D), jnp.float32),
    )
    return f(*xs, *ws, b.reshape(1, D))


def kernel(x_protocol, x_impression, x_treatment, edge_index_has,
           edge_index_suggests, edge_index_indicates, edge_index_issuggestedby,
           edge_weight_has, edge_weight_suggests, edge_weight_indicates,
           edge_weight_issuggestedby, params):
    lin = params["lin"]
    h = {
        "protocol": _mm([x_protocol], [lin["protocol"]["W"]],
                        lin["protocol"]["b"], "relu"),
        "impression": _mm([x_impression], [lin["impression"]["W"]],
                          lin["impression"]["b"], "relu"),
        "treatment": _mm([x_treatment], [lin["treatment"]["W"]],
                         lin["treatment"]["b"], "relu"),
    }
    edges = {
        "has": _pad_edges(edge_index_has, edge_weight_has),
        "suggests": _pad_edges(edge_index_suggests, edge_weight_suggests),
        "indicates": _pad_edges(edge_index_indicates, edge_weight_indicates),
        "issuggestedby": _pad_edges(edge_index_issuggestedby,
                                    edge_weight_issuggestedby),
    }
    seg = _make_segsum(edges["has"][3])
    src_of = {"has": "protocol", "suggests": "protocol",
              "indicates": "impression", "issuggestedby": "treatment"}
    for layer in params["convs"]:
        agg = {et: seg(h[src_of[et]], *edges[et][:3]) for et in edges}
        new_i = _mm([agg["has"], h["impression"]],
                    [layer["has"]["W_rel"], layer["has"]["W_root"]],
                    layer["has"]["b_rel"], "lrelu")
        new_t = _mm([agg["suggests"], h["treatment"]],
                    [layer["suggests"]["W_rel"], layer["suggests"]["W_root"]],
                    layer["suggests"]["b_rel"], "lrelu")
        new_p = _mm(
            [agg["indicates"], agg["issuggestedby"], h["protocol"]],
            [layer["indicates"]["W_rel"], layer["issuggestedby"]["W_rel"],
             layer["indicates"]["W_root"] + layer["issuggestedby"]["W_root"]],
            layer["indicates"]["b_rel"] + layer["issuggestedby"]["b_rel"],
            "lrelu")
        h = {"protocol": new_p, "impression": new_i, "treatment": new_t}
    return _mm([h["protocol"]], [params["out"]["W"]], params["out"]["b"], None)


# speedup vs baseline: 1.0920x; 1.0920x over previous
import functools

import jax
import jax.numpy as jnp
from jax import lax
from jax.experimental import pallas as pl
from jax.experimental.pallas import tpu as pltpu
from jax.experimental.pallas import tpu_sc as plsc

N = 50000
D = 128
NC = 2
NS = 16
BE = 128
SB = 2048
CH = 8448
NCHUNK = 6
RSTR = 51000
NREL = 4
NPASS = NREL * NCHUNK // NC
CPT = CH // NS
ZR = 66
BR = 1000


def _pad_edges(ei, w, src_off, dst_off):
    e = ei.shape[1]
    ept = ((e + NS * 2 * SB - 1) // (NS * 2 * SB)) * 2 * SB
    pad = ept * NS - e
    src = jnp.concatenate([ei[0] + src_off,
                           jnp.full((pad,), src_off, jnp.int32)])
    dst = jnp.concatenate([ei[1] + dst_off,
                           jnp.full((pad,), dst_off, jnp.int32)])
    wp = jnp.concatenate([w, jnp.zeros((pad,), jnp.float32)])
    return src, dst, wp, ept


@functools.cache
def _make_segsum(ept):
    nsb = ept // SB
    npair = nsb // 2
    nblk = SB // BE
    ep_rel = ept * NS
    mesh = plsc.VectorSubcoreMesh(core_axis_name="c", subcore_axis_name="s")

    @functools.partial(
        pl.kernel,
        mesh=mesh,
        compiler_params=pltpu.CompilerParams(needs_layout_passes=False),
        out_type=jax.ShapeDtypeStruct((NREL * RSTR, D), jnp.float32),
        scratch_types=[
            pltpu.VMEM_SHARED((CH, D), jnp.float32),
            pltpu.VMEM((ZR, D), jnp.float32),
            pltpu.VMEM((SB,), jnp.int32),
            pltpu.VMEM((SB,), jnp.int32),
            pltpu.VMEM((SB,), jnp.float32),
            pltpu.VMEM((SB,), jnp.int32),
            pltpu.VMEM((SB,), jnp.int32),
            pltpu.VMEM((SB,), jnp.float32),
            pltpu.VMEM((BE,), jnp.int32),
            pltpu.VMEM((BE,), jnp.int32),
            pltpu.VMEM((BE,), jnp.float32),
            pltpu.VMEM((BE,), jnp.int32),
            pltpu.VMEM((BE,), jnp.int32),
            pltpu.VMEM((BE,), jnp.float32),
            pltpu.VMEM((BE,), jnp.int32),
            pltpu.VMEM((BE,), jnp.int32),
            pltpu.VMEM((BE,), jnp.float32),
            pltpu.VMEM((BE,), jnp.int32),
            pltpu.VMEM((BE,), jnp.int32),
            pltpu.VMEM((BE,), jnp.float32),
            pltpu.VMEM((BE, D), jnp.float32),
            pltpu.VMEM((BE, D), jnp.float32),
            pltpu.SemaphoreType.DMA,
            pltpu.SemaphoreType.DMA,
            pltpu.SemaphoreType.DMA,
            pltpu.SemaphoreType.DMA,
            pltpu.SemaphoreType.DMA,
            pltpu.SemaphoreType.DMA,
        ],
    )
    def seg(h_hbm, src_hbm, dst_hbm, w_hbm, out_hbm,
            acc, zbuf, ebs0, ebd0, ebw0, ebs1, ebd1, ebw1,
            sts, std, stw, ovs, ovd, ovw,
            gs0, gd0, gw0, gs1, gd1, gw1, rows0, rows1,
            esem0, esem1, gsem0, gsem1, ssem0, ssem1):
        c = lax.axis_index("c")
        s = lax.axis_index("s")
        zero16f = jnp.zeros((16,), jnp.float32)
        zero16i = jnp.zeros((16,), jnp.int32)
        iota16 = lax.iota(jnp.int32, 16)

        def zb(i, carry):
            for u in range(8):
                zbuf[i, pl.ds(u * 16, 16)] = zero16f
            return carry

        lax.fori_loop(0, ZR, zb, 0)
        for g in range(8):
            sl = pl.ds(g * 16, 16)
            sts[sl] = zero16i
            std[sl] = zero16i
            stw[sl] = zero16f
            ovs[sl] = zero16i
            ovd[sl] = zero16i
            ovw[sl] = zero16f

        ebufs = ((ebs0, ebd0, ebw0, esem0), (ebs1, ebd1, ebw1, esem1))
        gsets = ((gs0, gd0, gw0, rows0, gsem0, ssem0),
                 (gs1, gd1, gw1, rows1, gsem1, ssem1))

        def start_gather(k):
            gs, gd, gw, rows, sem, ssem = gsets[k]
            pltpu.async_copy(h_hbm.at[gs], rows, sem)

        def wait_scatter(k):
            gs, gd, gw, rows, sem, ssem = gsets[k]
            pltpu.make_async_copy(rows, acc.at[gd], ssem).wait()

        def process(k):
            gs, gd, gw, rows, sem, ssem = gsets[k]
            pltpu.make_async_copy(h_hbm.at[gs], rows, sem).wait()

            def scale(j, carry2):
                wspl = plsc.load_gather(gw, [jnp.full((16,), j, jnp.int32)])
                for u in range(8):
                    sl2 = pl.ds(u * 16, 16)
                    rows[j, sl2] = rows[j, sl2] * wspl
                return carry2

            lax.fori_loop(0, BE, scale, 0)
            pltpu.async_copy(rows, acc.at[gd], ssem, add=True)

        def stage_to_set(k):
            gs, gd, gw, rows, sem, ssem = gsets[k]
            for g in range(8):
                sl = pl.ds(g * 16, 16)
                gs[sl] = sts[sl]
                gd[sl] = std[sl]
                gw[sl] = stw[sl]

        def ovf_to_stage():
            for g in range(8):
                sl = pl.ds(g * 16, 16)
                sts[sl] = ovs[sl]
                std[sl] = ovd[sl]
                stw[sl] = ovw[sl]

        def make_fire(k):
            o = 1 - k

            def fire_k(st):
                cnt, fp, pd0, pd1, ps0, ps1 = st
                pend_other = pd1 if k == 0 else pd0
                ps_k = ps0 if k == 0 else ps1

                @pl.when(pend_other == 1)
                def _():
                    process(o)

                @pl.when(ps_k == 1)
                def _():
                    wait_scatter(k)

                stage_to_set(k)
                start_gather(k)
                ovf_to_stage()
                one = jnp.int32(1)
                if k == 0:
                    return (cnt - BE, one, one, jnp.int32(0),
                            jnp.int32(0), ps1 | pd1)
                return (cnt - BE, jnp.int32(0), jnp.int32(0), one,
                        ps0 | pd0, jnp.int32(0))
            return fire_k

        fire0 = make_fire(0)
        fire1 = make_fire(1)

        def do_pass(p, pcarry):
            q = c * NPASS + p
            rel = q // NCHUNK
            qq = q - rel * NCHUNK
            base = rel * RSTR + qq * CH
            ebase = rel * ep_rel + s * ept

            def load_sb(t, half):
                bs, bd, bw, sem = ebufs[half]
                e0 = ebase + t * SB
                pltpu.async_copy(src_hbm.at[pl.ds(e0, SB)], bs, sem)
                pltpu.async_copy(dst_hbm.at[pl.ds(e0, SB)], bd, sem)
                pltpu.async_copy(w_hbm.at[pl.ds(e0, SB)], bw, sem)

            def wait_sb(t, half):
                bs, bd, bw, sem = ebufs[half]
                e0 = ebase + t * SB
                pltpu.make_async_copy(
                    src_hbm.at[pl.ds(e0, SB)], bs, sem).wait()
                pltpu.make_async_copy(
                    dst_hbm.at[pl.ds(e0, SB)], bd, sem).wait()
                pltpu.make_async_copy(
                    w_hbm.at[pl.ds(e0, SB)], bw, sem).wait()

            with jax.named_scope("segzero"):
                for k in range(CPT // ZR):
                    pltpu.sync_copy(zbuf, acc.at[pl.ds(s * CPT + k * ZR, ZR)])
                plsc.subcore_barrier()

            def scan_buf(half, st):
                bs, bd, bw, _ = ebufs[half]

                def eblk(b, st2):
                    cnt = st2[0]
                    boff = b * BE
                    for g in range(8):
                        sl = pl.ds(boff + g * 16, 16)
                        dv = bd[sl]
                        sv = bs[sl]
                        wv = bw[sl]
                        inm = (dv >= base) & (dv < base + CH)
                        ones = jnp.where(inm, 1, 0).astype(jnp.int32)
                        pos = cnt + plsc.cumsum(ones) - 1
                        posm = pos & (BE - 1)
                        in_a = inm & (pos < BE)
                        in_b = inm & (pos >= BE)
                        dloc = dv - base
                        plsc.store_scatter(sts, [posm], sv, mask=in_a)
                        plsc.store_scatter(std, [posm], dloc, mask=in_a)
                        plsc.store_scatter(stw, [posm], wv, mask=in_a)
                        plsc.store_scatter(ovs, [posm], sv, mask=in_b)
                        plsc.store_scatter(ovd, [posm], dloc, mask=in_b)
                        plsc.store_scatter(ovw, [posm], wv, mask=in_b)
                        cnt = cnt + plsc.all_reduce_population_count(inm)

                    st2 = (cnt,) + st2[1:]

                    def fire(stf):
                        return lax.cond(stf[1] == 0, fire0, fire1, stf)

                    return lax.cond(jnp.max(cnt) >= BE, fire,
                                    lambda a: a, st2)

                return lax.fori_loop(0, nblk, eblk, st)

            load_sb(0, 0)

            def sbpair(t2, st):
                t0 = 2 * t2
                load_sb(t0 + 1, 1)
                wait_sb(t0, 0)
                st = scan_buf(0, st)

                @pl.when(t2 + 1 < npair)
                def _():
                    load_sb(t0 + 2, 0)

                wait_sb(t0 + 1, 1)
                return scan_buf(1, st)

            z = jnp.int32(0)
            with jax.named_scope("segscan"):
                cnt, fp, pd0, pd1, ps0, ps1 = lax.fori_loop(
                    0, npair, sbpair, (zero16i, z, z, z, z, z))

            with jax.named_scope("segflush"):
                @pl.when(pd0 == 1)
                def _():
                    process(0)

                @pl.when(pd1 == 1)
                def _():
                    process(1)

                @pl.when((ps0 | pd0) == 1)
                def _():
                    wait_scatter(0)

                @pl.when((ps1 | pd1) == 1)
                def _():
                    wait_scatter(1)

                for g in range(8):
                    sl = pl.ds(g * 16, 16)
                    lane = iota16 + g * 16
                    stw[sl] = jnp.where(lane < cnt, stw[sl], 0.0)
                stage_to_set(0)
                start_gather(0)
                process(0)
                wait_scatter(0)
            with jax.named_scope("segout"):
                plsc.subcore_barrier()
                pltpu.sync_copy(
                    acc.at[pl.ds(s * CPT, CPT)],
                    out_hbm.at[pl.ds(base + s * CPT, CPT)])
                plsc.subcore_barrier()
            return pcarry

        lax.fori_loop(0, NPASS, do_pass, 0)

    return seg


def _proj3(x_all, w_stack, b_stack):
    def body(x_ref, w_ref, b_ref, o_ref):
        acc = jnp.dot(x_ref[...], w_ref[0],
                      preferred_element_type=jnp.float32) + b_ref[0]
        o_ref[...] = jnp.maximum(acc, 0.0)

    f = pl.pallas_call(
        body,
        grid=(3 * N // BR,),
        in_specs=[
            pl.BlockSpec((BR, D), lambda i: (i, 0)),
            pl.BlockSpec((1, D, D), lambda i: (i // 50, 0, 0)),
            pl.BlockSpec((1, 1, D), lambda i: (i // 50, 0, 0)),
        ],
        out_specs=pl.BlockSpec((BR, D), lambda i: (i, 0)),
        out_shape=jax.ShapeDtypeStruct((3 * N, D), jnp.float32),
    )
    return f(x_all, w_stack, b_stack)


def _conv3(agg_all, h_all, w1, w2, wroot, b_stack):
    def body(a1_ref, a2_ref, h_ref, w1_ref, w2_ref, wr_ref, b_ref, o_ref):
        acc = jnp.dot(a1_ref[...], w1_ref[0],
                      preferred_element_type=jnp.float32)
        acc = acc + jnp.dot(a2_ref[...], w2_ref[0],
                            preferred_element_type=jnp.float32)
        acc = acc + jnp.dot(h_ref[...], wr_ref[0],
                            preferred_element_type=jnp.float32)
        acc = acc + b_ref[0]
        o_ref[...] = jnp.where(acc > 0, acc, acc * 0.01)

    nbr = RSTR // BR
    f = pl.pallas_call(
        body,
        grid=(3 * N // BR,),
        in_specs=[
            pl.BlockSpec((BR, D),
                         lambda i: (((i // 50 + 2) % 3) * nbr + i % 50, 0)),
            pl.BlockSpec((BR, D), lambda i: (3 * nbr + i % 50, 0)),
            pl.BlockSpec((BR, D), lambda i: (i, 0)),
            pl.BlockSpec((1, D, D), lambda i: (i // 50, 0, 0)),
            pl.BlockSpec((1, D, D), lambda i: (i // 50, 0, 0)),
            pl.BlockSpec((1, D, D), lambda i: (i // 50, 0, 0)),
            pl.BlockSpec((1, 1, D), lambda i: (i // 50, 0, 0)),
        ],
        out_specs=pl.BlockSpec((BR, D), lambda i: (i, 0)),
        out_shape=jax.ShapeDtypeStruct((3 * N, D), jnp.float32),
    )
    return f(agg_all, agg_all, h_all, w1, w2, wroot, b_stack)


def _proj_out(h_all, w, b):
    def body(x_ref, w_ref, b_ref, o_ref):
        o_ref[...] = jnp.dot(x_ref[...], w_ref[...],
                             preferred_element_type=jnp.float32) + b_ref[...]

    f = pl.pallas_call(
        body,
        grid=(N // BR,),
        in_specs=[
            pl.BlockSpec((BR, D), lambda i: (i, 0)),
            pl.BlockSpec((D, D), lambda i: (0, 0)),
            pl.BlockSpec((1, D), lambda i: (0, 0)),
        ],
        out_specs=pl.BlockSpec((BR, D), lambda i: (i, 0)),
        out_shape=jax.ShapeDtypeStruct((N, D), jnp.float32),
    )
    return f(h_all, w, b.reshape(1, D))


def kernel(x_protocol, x_impression, x_treatment, edge_index_has,
           edge_index_suggests, edge_index_indicates, edge_index_issuggestedby,
           edge_weight_has, edge_weight_suggests, edge_weight_indicates,
           edge_weight_issuggestedby, params):
    lin = params["lin"]
    x_all = jnp.concatenate([x_protocol, x_impression, x_treatment])
    w_lin = jnp.stack([lin[t]["W"] for t in
                       ("protocol", "impression", "treatment")])
    b_lin = jnp.stack([lin[t]["b"].reshape(1, D) for t in
                       ("protocol", "impression", "treatment")])
    h_all = _proj3(x_all, w_lin, b_lin)

    rels = [
        (edge_index_has, edge_weight_has, 0),
        (edge_index_suggests, edge_weight_suggests, 0),
        (edge_index_indicates, edge_weight_indicates, N),
        (edge_index_issuggestedby, edge_weight_issuggestedby, 2 * N),
    ]
    srcs, dsts, ws = [], [], []
    ept = None
    for r, (ei, ew, soff) in enumerate(rels):
        sp, dp, wp, ept = _pad_edges(ei, ew, soff, r * RSTR)
        srcs.append(sp)
        dsts.append(dp)
        ws.append(wp)
    src_all = jnp.concatenate(srcs)
    dst_all = jnp.concatenate(dsts)
    w_all = jnp.concatenate(ws)
    seg = _make_segsum(ept)

    zero_w = jnp.zeros((D, D), jnp.float32)
    for layer in params["convs"]:
        agg_all = seg(h_all, src_all, dst_all, w_all)
        w1 = jnp.stack([layer["indicates"]["W_rel"],
                        layer["has"]["W_rel"],
                        layer["suggests"]["W_rel"]])
        w2 = jnp.stack([layer["issuggestedby"]["W_rel"], zero_w, zero_w])
        wroot = jnp.stack([
            layer["indicates"]["W_root"] + layer["issuggestedby"]["W_root"],
            layer["has"]["W_root"],
            layer["suggests"]["W_root"]])
        b_stack = jnp.stack([
            (layer["indicates"]["b_rel"]
             + layer["issuggestedby"]["b_rel"]).reshape(1, D),
            layer["has"]["b_rel"].reshape(1, D),
            layer["suggests"]["b_rel"].reshape(1, D)])
        h_all = _conv3(agg_all, h_all, w1, w2, wroot, b_stack)
    return _proj_out(h_all, params["out"]["W"], params["out"]["b"])


# speedup vs baseline: 1.2560x; 1.1502x over previous
import functools

import jax
import jax.numpy as jnp
from jax import lax
from jax.experimental import pallas as pl
from jax.experimental.pallas import tpu as pltpu
from jax.experimental.pallas import tpu_sc as plsc

N = 50000
D = 128
NC = 2
NS = 16
BE = 128
CH = 8448
NCHUNK = 6
NPAD = CH * NCHUNK
PASSES = NCHUNK // NC
CPT = CH // NS
ZR = 66
BR = 1000


def _pad_edges(ei, w):
    e = ei.shape[1]
    ept = ((e + NS * BE - 1) // (NS * BE)) * BE
    pad = ept * NS - e
    src = jnp.concatenate([ei[0], jnp.zeros((pad,), jnp.int32)])
    dst = jnp.concatenate([ei[1], jnp.zeros((pad,), jnp.int32)])
    wp = jnp.concatenate([w, jnp.zeros((pad,), jnp.float32)])
    return src, dst, wp, ept


@functools.cache
def _make_segsum(ept):
    nblk = ept // BE
    mesh = plsc.VectorSubcoreMesh(core_axis_name="c", subcore_axis_name="s")

    @functools.partial(
        pl.kernel,
        mesh=mesh,
        compiler_params=pltpu.CompilerParams(needs_layout_passes=False),
        out_type=jax.ShapeDtypeStruct((NPAD, D), jnp.float32),
        scratch_types=[
            pltpu.VMEM_SHARED((CH, D), jnp.float32),
            pltpu.VMEM((ZR, D), jnp.float32),
            pltpu.VMEM((BE,), jnp.int32),
            pltpu.VMEM((BE,), jnp.int32),
            pltpu.VMEM((BE,), jnp.float32),
            pltpu.VMEM((BE,), jnp.int32),
            pltpu.VMEM((BE,), jnp.int32),
            pltpu.VMEM((BE,), jnp.int32),
            pltpu.VMEM((BE,), jnp.int32),
            pltpu.VMEM((BE,), jnp.float32),
            pltpu.VMEM((BE,), jnp.float32),
            pltpu.VMEM((BE, D), jnp.float32),
            pltpu.SemaphoreType.DMA,
        ],
    )
    def seg(h_hbm, src_hbm, dst_hbm, w_hbm, out_hbm,
            acc, zbuf, sbuf, dbuf, wbuf,
            csA, csB, cdA, cdB, cwA, cwB, rows, sem):
        c = lax.axis_index("c")
        s = lax.axis_index("s")
        zero16f = jnp.zeros((16,), jnp.float32)
        zero16i = jnp.zeros((16,), jnp.int32)
        iota16 = lax.iota(jnp.int32, 16)

        def zb(i, carry):
            for u in range(8):
                zbuf[i, pl.ds(u * 16, 16)] = zero16f
            return carry

        lax.fori_loop(0, ZR, zb, 0)
        for g in range(8):
            sl = pl.ds(g * 16, 16)
            csA[sl] = zero16i
            csB[sl] = zero16i
            cdA[sl] = zero16i
            cdB[sl] = zero16i
            cwA[sl] = zero16f
            cwB[sl] = zero16f

        def fire_batch():
            pltpu.async_copy(h_hbm.at[csA], rows, sem).wait()

            def scale(j, carry2):
                wspl = plsc.load_gather(cwA, [jnp.full((16,), j, jnp.int32)])
                for u in range(8):
                    sl2 = pl.ds(u * 16, 16)
                    rows[j, sl2] = rows[j, sl2] * wspl
                return carry2

            lax.fori_loop(0, BE, scale, 0)
            pltpu.sync_copy(rows, acc.at[cdA], add=True)

        e0t = s * ept
        for p in range(PASSES):
            q = c * PASSES + p
            base = q * CH
            for k in range(CPT // ZR):
                pltpu.sync_copy(zbuf, acc.at[pl.ds(s * CPT + k * ZR, ZR)])
            plsc.subcore_barrier()

            def eblk(i, cnt):
                e0 = e0t + i * BE
                pltpu.sync_copy(src_hbm.at[pl.ds(e0, BE)], sbuf)
                pltpu.sync_copy(dst_hbm.at[pl.ds(e0, BE)], dbuf)
                pltpu.sync_copy(w_hbm.at[pl.ds(e0, BE)], wbuf)
                for g in range(8):
                    sl = pl.ds(g * 16, 16)
                    dv = dbuf[sl]
                    sv = sbuf[sl]
                    wv = wbuf[sl]
                    inm = (dv >= base) & (dv < base + CH)
                    ones = jnp.where(inm, 1, 0).astype(jnp.int32)
                    pos = cnt + plsc.cumsum(ones) - 1
                    posm = pos & (BE - 1)
                    in_a = inm & (pos < BE)
                    in_b = inm & (pos >= BE)
                    dloc = dv - base
                    plsc.store_scatter(csA, [posm], sv, mask=in_a)
                    plsc.store_scatter(csB, [posm], sv, mask=in_b)
                    plsc.store_scatter(cdA, [posm], dloc, mask=in_a)
                    plsc.store_scatter(cdB, [posm], dloc, mask=in_b)
                    plsc.store_scatter(cwA, [posm], wv, mask=in_a)
                    plsc.store_scatter(cwB, [posm], wv, mask=in_b)
                    cnt = cnt + plsc.all_reduce_population_count(inm)

                def fire(cv):
                    fire_batch()
                    for g2 in range(8):
                        sl2 = pl.ds(g2 * 16, 16)
                        csA[sl2] = csB[sl2]
                        cdA[sl2] = cdB[sl2]
                        cwA[sl2] = cwB[sl2]
                    return cv - BE

                cnt = lax.cond(jnp.max(cnt) >= BE, fire, lambda cv: cv, cnt)
                return cnt

            cnt = lax.fori_loop(0, nblk, eblk, zero16i)
            for g in range(8):
                sl = pl.ds(g * 16, 16)
                lane = iota16 + g * 16
                cwA[sl] = jnp.where(lane < cnt, cwA[sl], 0.0)
            fire_batch()
            plsc.subcore_barrier()
            pltpu.sync_copy(
                acc.at[pl.ds(s * CPT, CPT)],
                out_hbm.at[pl.ds(base + s * CPT, CPT)])
            plsc.subcore_barrier()

    return seg


def _mm(xs, ws, b, act):
    nin = len(xs)

    def body(*refs):
        in_refs = refs[:nin]
        w_refs = refs[nin:2 * nin]
        b_ref = refs[2 * nin]
        o_ref = refs[2 * nin + 1]
        acc = jnp.zeros((BR, D), jnp.float32)
        for xr, wr in zip(in_refs, w_refs):
            acc = acc + jnp.dot(xr[...], wr[...],
                                preferred_element_type=jnp.float32)
        acc = acc + b_ref[...]
        if act == "relu":
            acc = jnp.maximum(acc, 0.0)
        elif act == "lrelu":
            acc = jnp.where(acc > 0, acc, acc * 0.01)
        o_ref[...] = acc

    in_specs = (
        [pl.BlockSpec((BR, D), lambda i: (i, 0)) for _ in xs]
        + [pl.BlockSpec((D, D), lambda i: (0, 0)) for _ in ws]
        + [pl.BlockSpec((1, D), lambda i: (0, 0))]
    )
    f = pl.pallas_call(
        body,
        grid=(N // BR,),
        in_specs=in_specs,
        out_specs=pl.BlockSpec((BR, D), lambda i: (i, 0)),
        out_shape=jax.ShapeDtypeStruct((N, D), jnp.float32),
    )
    return f(*xs, *ws, b.reshape(1, D))


def kernel(x_protocol, x_impression, x_treatment, edge_index_has,
           edge_index_suggests, edge_index_indicates, edge_index_issuggestedby,
           edge_weight_has, edge_weight_suggests, edge_weight_indicates,
           edge_weight_issuggestedby, params):
    lin = params["lin"]
    h = {
        "protocol": _mm([x_protocol], [lin["protocol"]["W"]],
                        lin["protocol"]["b"], "relu"),
        "impression": _mm([x_impression], [lin["impression"]["W"]],
                          lin["impression"]["b"], "relu"),
        "treatment": _mm([x_treatment], [lin["treatment"]["W"]],
                         lin["treatment"]["b"], "relu"),
    }
    edges = {
        "has": _pad_edges(edge_index_has, edge_weight_has),
        "suggests": _pad_edges(edge_index_suggests, edge_weight_suggests),
        "indicates": _pad_edges(edge_index_indicates, edge_weight_indicates),
        "issuggestedby": _pad_edges(edge_index_issuggestedby,
                                    edge_weight_issuggestedby),
    }
    seg = _make_segsum(edges["has"][3])
    src_of = {"has": "protocol", "suggests": "protocol",
              "indicates": "impression", "issuggestedby": "treatment"}
    for layer in params["convs"]:
        agg = {et: seg(h[src_of[et]], *edges[et][:3]) for et in edges}
        new_i = _mm([agg["has"], h["impression"]],
                    [layer["has"]["W_rel"], layer["has"]["W_root"]],
                    layer["has"]["b_rel"], "lrelu")
        new_t = _mm([agg["suggests"], h["treatment"]],
                    [layer["suggests"]["W_rel"], layer["suggests"]["W_root"]],
                    layer["suggests"]["b_rel"], "lrelu")
        new_p = _mm(
            [agg["indicates"], agg["issuggestedby"], h["protocol"]],
            [layer["indicates"]["W_rel"], layer["issuggestedby"]["W_rel"],
             layer["indicates"]["W_root"] + layer["issuggestedby"]["W_root"]],
            layer["indicates"]["b_rel"] + layer["issuggestedby"]["b_rel"],
            "lrelu")
        h = {"protocol": new_p, "impression": new_i, "treatment": new_t}
    return _mm([h["protocol"]], [params["out"]["W"]], params["out"]["b"], None)
